# baseline (speedup 1.0000x reference)
import functools

import jax
import jax.numpy as jnp
from jax import lax
from jax.experimental import pallas as pl
from jax.experimental.pallas import tpu as pltpu
from jax.experimental.pallas import tpu_sc as plsc

N = 10000
E = 320000
D = 128
G = 64

NT = 16
K = 128
CH = 158
ET = CH * K
EP = NT * ET
NP = NT * 640
DUMMY = N + 200


_DW = 16


def _deg_body(sidx_s, sidx_t, ones_hbm, zsrc, deg_o, deg_i, sidx_v, ones_v, acc):
    cid = lax.axis_index("c")
    sid = lax.axis_index("s")

    def run(sidx_hbm, out_hbm):
        pltpu.sync_copy(sidx_hbm.at[sid], sidx_v)
        pltpu.sync_copy(ones_hbm, ones_v)
        pltpu.sync_copy(zsrc, acc.at[pl.ds(sid * 640, 640)])
        plsc.subcore_barrier()

        def body(j, c):
            pltpu.sync_copy(ones_v, acc.at[sidx_v.at[j]], add=True)
            return c
        lax.fori_loop(0, CH, body, 0)

        plsc.subcore_barrier()
        pltpu.sync_copy(acc.at[pl.ds(sid * 640, 640)],
                        out_hbm.at[pl.ds(sid * 640, 640)])

    @pl.when(cid == 0)
    def _():
        run(sidx_s, deg_o)

    @pl.when(cid == 1)
    def _():
        run(sidx_t, deg_i)


@functools.cache
def _get_sc_degrees():
    mesh = plsc.VectorSubcoreMesh(core_axis_name="c", subcore_axis_name="s")
    return pl.kernel(
        _deg_body,
        out_type=[jax.ShapeDtypeStruct((NP, _DW), jnp.float32),
                  jax.ShapeDtypeStruct((NP, _DW), jnp.float32)],
        mesh=mesh,
        scratch_types=[
            pltpu.VMEM((CH, K), jnp.int32),
            pltpu.VMEM((K, _DW), jnp.float32),
            pltpu.VMEM_SHARED((NP, _DW), jnp.float32),
        ],
        compiler_params=pltpu.CompilerParams(use_tc_tiling_on_sc=False),
    )


DH = D // 2


def _agg_body(hs0, hs1, ht0, ht1, gidx_s, sidx_s, gidx_t, sidx_t, zsrc,
              s0_out, s1_out, t0_out, t1_out,
              gidx_v, sidx_v, rows0, rows1, acc, gsem0, gsem1, ssem0, ssem1):
    cid = lax.axis_index("c")
    sid = lax.axis_index("s")

    def run(feat_hbm, out_hbm):
        pltpu.sync_copy(zsrc, acc.at[pl.ds(sid * 640, 640)])
        plsc.subcore_barrier()

        pltpu.async_copy(feat_hbm.at[gidx_v.at[0]], rows0, gsem0)
        pltpu.async_copy(feat_hbm.at[gidx_v.at[1]], rows1, gsem1)

        def body(j2, c):
            j = j2 * 2
            pltpu.make_async_copy(feat_hbm.at[pl.ds(0, K)], rows0, gsem0).wait()
            pltpu.async_copy(rows0, acc.at[sidx_v.at[j]], ssem0, add=True)
            pltpu.make_async_copy(feat_hbm.at[pl.ds(0, K)], rows1, gsem1).wait()
            pltpu.async_copy(rows1, acc.at[sidx_v.at[j + 1]], ssem1, add=True)
            pltpu.make_async_copy(rows0, acc.at[sidx_v.at[0]], ssem0).wait()

            @pl.when(j + 2 < CH)
            def _():
                pltpu.async_copy(feat_hbm.at[gidx_v.at[j + 2]], rows0, gsem0)

            pltpu.make_async_copy(rows1, acc.at[sidx_v.at[0]], ssem1).wait()

            @pl.when(j + 3 < CH)
            def _():
                pltpu.async_copy(feat_hbm.at[gidx_v.at[j + 3]], rows1, gsem1)
            return c

        lax.fori_loop(0, CH // 2, body, 0)
        plsc.subcore_barrier()
        pltpu.sync_copy(acc.at[pl.ds(sid * 640, 640)],
                        out_hbm.at[pl.ds(sid * 640, 640)])
        plsc.subcore_barrier()

    def direction(gidx_hbm, sidx_hbm, f0, f1, o0, o1):
        pltpu.sync_copy(gidx_hbm.at[sid], gidx_v)
        pltpu.sync_copy(sidx_hbm.at[sid], sidx_v)
        run(f0, o0)
        run(f1, o1)

    @pl.when(cid == 0)
    def _():
        direction(gidx_s, sidx_s, hs0, hs1, s0_out, s1_out)

    @pl.when(cid == 1)
    def _():
        direction(gidx_t, sidx_t, ht0, ht1, t0_out, t1_out)


@functools.cache
def _get_sc_agg():
    mesh = plsc.VectorSubcoreMesh(core_axis_name="c", subcore_axis_name="s")
    return pl.kernel(
        _agg_body,
        out_type=[jax.ShapeDtypeStruct((NP, DH), jnp.float32)] * 4,
        mesh=mesh,
        scratch_types=[
            pltpu.VMEM((CH, K), jnp.int32),
            pltpu.VMEM((CH, K), jnp.int32),
            pltpu.VMEM((K, DH), jnp.float32),
            pltpu.VMEM((K, DH), jnp.float32),
            pltpu.VMEM_SHARED((NP, DH), jnp.float32),
            pltpu.SemaphoreType.DMA,
            pltpu.SemaphoreType.DMA,
            pltpu.SemaphoreType.DMA,
            pltpu.SemaphoreType.DMA,
        ],
        compiler_params=pltpu.CompilerParams(use_tc_tiling_on_sc=False),
    )


_BR = 1000


def _prep_body(dgo_ref, dgi_ref, x_ref, io_ref, ii_ref,
               xs0_ref, xs1_ref, xt0_ref, xt1_ref):
    do = dgo_ref[...]
    di = dgi_ref[...]
    io = jnp.where(do > 0, lax.rsqrt(do), 0.0)
    ii = jnp.where(di > 0, lax.rsqrt(di), 0.0)
    io_ref[...] = io
    ii_ref[...] = ii
    x = x_ref[...]
    xs0_ref[...] = ii * x[:, :DH]
    xs1_ref[...] = ii * x[:, DH:]
    xt0_ref[...] = io * x[:, :DH]
    xt1_ref[...] = io * x[:, DH:]


_tc_prep = pl.pallas_call(
    _prep_body,
    grid=(N // _BR,),
    in_specs=[pl.BlockSpec((_BR, 1), lambda i: (i, 0)),
              pl.BlockSpec((_BR, 1), lambda i: (i, 0)),
              pl.BlockSpec((_BR, D), lambda i: (i, 0))],
    out_specs=[pl.BlockSpec((_BR, 1), lambda i: (i, 0)),
               pl.BlockSpec((_BR, 1), lambda i: (i, 0))]
              + [pl.BlockSpec((_BR, DH), lambda i: (i, 0))] * 4,
    out_shape=[jax.ShapeDtypeStruct((N, 1), jnp.float32),
               jax.ShapeDtypeStruct((N, 1), jnp.float32)]
              + [jax.ShapeDtypeStruct((N, DH), jnp.float32)] * 4,
)


def _layer_body(scale_out, s0_ref, s1_ref, t0_ref, t1_ref, io_ref, ii_ref,
                ws0_ref, ws1_ref, wd0_ref, wd1_ref, bs_ref, bd_ref, *outs):
    io = io_ref[...]
    ii = ii_ref[...]
    f32 = jnp.float32
    hs_part = (jnp.dot(io * s0_ref[...], ws0_ref[...], preferred_element_type=f32)
               + jnp.dot(io * s1_ref[...], ws1_ref[...], preferred_element_type=f32)
               + bs_ref[...])
    hd_part = (jnp.dot(ii * t0_ref[...], wd0_ref[...], preferred_element_type=f32)
               + jnp.dot(ii * t1_ref[...], wd1_ref[...], preferred_element_type=f32)
               + bd_ref[...])
    h = jnp.maximum(0.5 * hs_part + 0.5 * hd_part, 0.0)
    if scale_out:
        outs[0][...] = ii * h[:, :DH]
        outs[1][...] = ii * h[:, DH:]
        outs[2][...] = io * h[:, :DH]
        outs[3][...] = io * h[:, DH:]
    else:
        outs[0][...] = h


def _make_layer(scale_out):
    if scale_out:
        out_specs = [pl.BlockSpec((_BR, DH), lambda i: (i, 0))] * 4
        out_shape = [jax.ShapeDtypeStruct((N, DH), jnp.float32)] * 4
    else:
        out_specs = [pl.BlockSpec((_BR, D), lambda i: (i, 0))]
        out_shape = [jax.ShapeDtypeStruct((N, D), jnp.float32)]
    return pl.pallas_call(
        functools.partial(_layer_body, scale_out),
        grid=(N // _BR,),
        in_specs=[pl.BlockSpec((_BR, DH), lambda i: (i, 0))] * 4
                 + [pl.BlockSpec((_BR, 1), lambda i: (i, 0)),
                    pl.BlockSpec((_BR, 1), lambda i: (i, 0))]
                 + [pl.BlockSpec((DH, D), lambda i: (0, 0))] * 4
                 + [pl.BlockSpec((1, D), lambda i: (0, 0)),
                    pl.BlockSpec((1, D), lambda i: (0, 0))],
        out_specs=out_specs,
        out_shape=out_shape,
    )


_tc_layer_mid = _make_layer(True)
_tc_layer_last = _make_layer(False)

_PBR = 200


def _pool_body(h_ref, b_ref, wl1_ref, bl1_ref, wl2_ref, bl2_ref,
               out_ref, pooled_ref):
    pid = pl.program_id(0)
    ninf = jnp.float32(-jnp.inf)

    @pl.when(pid == 0)
    def _():
        pooled_ref[...] = jnp.full((G, D), ninf, jnp.float32)

    hb = h_ref[...]
    bb = b_ref[...]
    gids = lax.broadcasted_iota(jnp.int32, (G, 1, 1), 0)
    mask = bb[None, :, :] == gids
    v = jnp.where(mask, hb[None, :, :], ninf)
    pooled_ref[...] = jnp.maximum(pooled_ref[...], jnp.max(v, axis=1))

    @pl.when(pid == N // _PBR - 1)
    def _():
        p = pooled_ref[...]
        z = jnp.dot(p, wl1_ref[...], preferred_element_type=jnp.float32) \
            + bl1_ref[...]
        z = jnp.maximum(z, 0.0)
        out_ref[...] = jnp.dot(z, wl2_ref[...],
                               preferred_element_type=jnp.float32) + bl2_ref[...]


_tc_pool = pl.pallas_call(
    _pool_body,
    grid=(N // _PBR,),
    in_specs=[pl.BlockSpec((_PBR, D), lambda i: (i, 0)),
              pl.BlockSpec((_PBR, 1), lambda i: (i, 0)),
              pl.BlockSpec((D, D), lambda i: (0, 0)),
              pl.BlockSpec((1, D), lambda i: (0, 0)),
              pl.BlockSpec((D, D), lambda i: (0, 0)),
              pl.BlockSpec((1, D), lambda i: (0, 0))],
    out_specs=pl.BlockSpec((G, D), lambda i: (0, 0)),
    out_shape=jax.ShapeDtypeStruct((G, D), jnp.float32),
    scratch_shapes=[pltpu.VMEM((G, D), jnp.float32)],
)


def _pad_idx(a, fill):
    pad = jnp.full((EP - E,), fill, jnp.int32)
    return jnp.concatenate([a, pad]).reshape(NT, CH, K)


def kernel(x, edge_index, batch, W1s, b1s, W1d, b1d, W2s, b2s, W2d, b2d,
           W3s, b3s, W3d, b3d, Wl1, bl1, Wl2, bl2):
    row = edge_index[0]
    col = edge_index[1]
    gidx_s = _pad_idx(col, 0)
    sidx_s = _pad_idx(row, DUMMY)
    gidx_t = _pad_idx(row, 0)
    sidx_t = _pad_idx(col, DUMMY)
    zsrc = jnp.zeros((640, DH), jnp.float32)
    ones_rows = jnp.ones((K, _DW), jnp.float32)
    zsrc_d = jnp.zeros((640, _DW), jnp.float32)

    deg_o_p, deg_i_p = _get_sc_degrees()(sidx_s, sidx_t, ones_rows, zsrc_d)
    deg_o = deg_o_p[:N, :1]
    deg_i = deg_i_p[:N, :1]

    io, ii, hs0, hs1, ht0, ht1 = _tc_prep(deg_o, deg_i, x)

    weights = [(W1s, b1s, W1d, b1d), (W2s, b2s, W2d, b2d), (W3s, b3s, W3d, b3d)]
    for l, (ws, bs, wd, bd) in enumerate(weights):
        s0, s1, t0, t1 = _get_sc_agg()(
            hs0, hs1, ht0, ht1, gidx_s, sidx_s, gidx_t, sidx_t, zsrc)
        args = (s0, s1, t0, t1, io, ii, ws[:DH], ws[DH:], wd[:DH], wd[DH:],
                bs.reshape(1, D), bd.reshape(1, D))
        if l < 2:
            hs0, hs1, ht0, ht1 = _tc_layer_mid(*args)
        else:
            h = _tc_layer_last(*args)[0]

    wl1p = jnp.zeros((D, D), jnp.float32).at[:, :5].set(Wl1)
    bl1p = jnp.zeros((1, D), jnp.float32).at[:, :5].set(bl1[None, :])
    wl2p = jnp.zeros((D, D), jnp.float32).at[:5, :1].set(Wl2)
    bl2p = jnp.zeros((1, D), jnp.float32).at[:, :1].set(bl2[None, :])

    out = _tc_pool(h, batch.reshape(N, 1).astype(jnp.int32),
                   wl1p, bl1p, wl2p, bl2p)
    return out[:, :1]


# speedup vs baseline: 1.1363x; 1.1363x over previous
import functools

import jax
import jax.numpy as jnp
from jax import lax
from jax.experimental import pallas as pl
from jax.experimental.pallas import tpu as pltpu
from jax.experimental.pallas import tpu_sc as plsc

N = 10000
E = 320000
D = 128
G = 64

NT = 16
K = 128
CH = 158
ET = CH * K
EP = NT * ET
NP = NT * 640
DUMMY = N + 200


_DW = 16


def _deg_body(sidx_s, sidx_t, ones_hbm, zsrc, deg_o, deg_i, sidx_v, ones_v, acc):
    cid = lax.axis_index("c")
    sid = lax.axis_index("s")

    def run(sidx_hbm, out_hbm):
        pltpu.sync_copy(sidx_hbm.at[sid], sidx_v)
        pltpu.sync_copy(ones_hbm, ones_v)
        pltpu.sync_copy(zsrc, acc.at[pl.ds(sid * 640, 640)])
        plsc.subcore_barrier()

        def body(j, c):
            pltpu.sync_copy(ones_v, acc.at[sidx_v.at[j]], add=True)
            return c
        lax.fori_loop(0, CH, body, 0)

        plsc.subcore_barrier()
        pltpu.sync_copy(acc.at[pl.ds(sid * 640, 640)],
                        out_hbm.at[pl.ds(sid * 640, 640)])

    @pl.when(cid == 0)
    def _():
        run(sidx_s, deg_o)

    @pl.when(cid == 1)
    def _():
        run(sidx_t, deg_i)


@functools.cache
def _get_sc_degrees():
    mesh = plsc.VectorSubcoreMesh(core_axis_name="c", subcore_axis_name="s")
    return pl.kernel(
        _deg_body,
        out_type=[jax.ShapeDtypeStruct((NP, _DW), jnp.float32),
                  jax.ShapeDtypeStruct((NP, _DW), jnp.float32)],
        mesh=mesh,
        scratch_types=[
            pltpu.VMEM((CH, K), jnp.int32),
            pltpu.VMEM((K, _DW), jnp.float32),
            pltpu.VMEM_SHARED((NP, _DW), jnp.float32),
        ],
        compiler_params=pltpu.CompilerParams(use_tc_tiling_on_sc=False),
    )


DH = D // 2


def _agg_body(hs0, hs1, ht0, ht1, gidx_s, sidx_s, gidx_t, sidx_t, zsrc,
              s0_out, s1_out, t0_out, t1_out,
              gidx_v, sidx_v, rows0, rows1, acc, gsem0, gsem1):
    cid = lax.axis_index("c")
    sid = lax.axis_index("s")

    def run(feat_hbm, out_hbm):
        pltpu.sync_copy(zsrc, acc.at[pl.ds(sid * 640, 640)])
        plsc.subcore_barrier()

        pltpu.async_copy(feat_hbm.at[gidx_v.at[0]], rows0, gsem0)

        def body(j2, c):
            j = j2 * 2
            pltpu.async_copy(feat_hbm.at[gidx_v.at[j + 1]], rows1, gsem1)
            pltpu.make_async_copy(feat_hbm.at[pl.ds(0, K)], rows0, gsem0).wait()
            pltpu.sync_copy(rows0, acc.at[sidx_v.at[j]], add=True)

            @pl.when(j + 2 < CH)
            def _():
                pltpu.async_copy(feat_hbm.at[gidx_v.at[j + 2]], rows0, gsem0)

            pltpu.make_async_copy(feat_hbm.at[pl.ds(0, K)], rows1, gsem1).wait()
            pltpu.sync_copy(rows1, acc.at[sidx_v.at[j + 1]], add=True)
            return c

        lax.fori_loop(0, CH // 2, body, 0)
        plsc.subcore_barrier()
        pltpu.sync_copy(acc.at[pl.ds(sid * 640, 640)],
                        out_hbm.at[pl.ds(sid * 640, 640)])
        plsc.subcore_barrier()

    def direction(gidx_hbm, sidx_hbm, f0, f1, o0, o1):
        pltpu.sync_copy(gidx_hbm.at[sid], gidx_v)
        pltpu.sync_copy(sidx_hbm.at[sid], sidx_v)
        run(f0, o0)
        run(f1, o1)

    @pl.when(cid == 0)
    def _():
        direction(gidx_s, sidx_s, hs0, hs1, s0_out, s1_out)

    @pl.when(cid == 1)
    def _():
        direction(gidx_t, sidx_t, ht0, ht1, t0_out, t1_out)


@functools.cache
def _get_sc_agg():
    mesh = plsc.VectorSubcoreMesh(core_axis_name="c", subcore_axis_name="s")
    return pl.kernel(
        _agg_body,
        out_type=[jax.ShapeDtypeStruct((NP, DH), jnp.float32)] * 4,
        mesh=mesh,
        scratch_types=[
            pltpu.VMEM((CH, K), jnp.int32),
            pltpu.VMEM((CH, K), jnp.int32),
            pltpu.VMEM((K, DH), jnp.float32),
            pltpu.VMEM((K, DH), jnp.float32),
            pltpu.VMEM_SHARED((NP, DH), jnp.float32),
            pltpu.SemaphoreType.DMA,
            pltpu.SemaphoreType.DMA,
        ],
        compiler_params=pltpu.CompilerParams(use_tc_tiling_on_sc=False),
    )


_BR = 1000


def _prep_body(dgo_ref, dgi_ref, x_ref, io_ref, ii_ref,
               xs0_ref, xs1_ref, xt0_ref, xt1_ref):
    do = dgo_ref[...]
    di = dgi_ref[...]
    io = jnp.where(do > 0, lax.rsqrt(do), 0.0)
    ii = jnp.where(di > 0, lax.rsqrt(di), 0.0)
    io_ref[...] = io
    ii_ref[...] = ii
    x = x_ref[...]
    xs0_ref[...] = ii * x[:, :DH]
    xs1_ref[...] = ii * x[:, DH:]
    xt0_ref[...] = io * x[:, :DH]
    xt1_ref[...] = io * x[:, DH:]


_tc_prep = pl.pallas_call(
    _prep_body,
    grid=(N // _BR,),
    in_specs=[pl.BlockSpec((_BR, 1), lambda i: (i, 0)),
              pl.BlockSpec((_BR, 1), lambda i: (i, 0)),
              pl.BlockSpec((_BR, D), lambda i: (i, 0))],
    out_specs=[pl.BlockSpec((_BR, 1), lambda i: (i, 0)),
               pl.BlockSpec((_BR, 1), lambda i: (i, 0))]
              + [pl.BlockSpec((_BR, DH), lambda i: (i, 0))] * 4,
    out_shape=[jax.ShapeDtypeStruct((N, 1), jnp.float32),
               jax.ShapeDtypeStruct((N, 1), jnp.float32)]
              + [jax.ShapeDtypeStruct((N, DH), jnp.float32)] * 4,
)


def _layer_body(scale_out, s0_ref, s1_ref, t0_ref, t1_ref, io_ref, ii_ref,
                ws0_ref, ws1_ref, wd0_ref, wd1_ref, bs_ref, bd_ref, *outs):
    io = io_ref[...]
    ii = ii_ref[...]
    f32 = jnp.float32
    hs_part = (jnp.dot(io * s0_ref[...], ws0_ref[...], preferred_element_type=f32)
               + jnp.dot(io * s1_ref[...], ws1_ref[...], preferred_element_type=f32)
               + bs_ref[...])
    hd_part = (jnp.dot(ii * t0_ref[...], wd0_ref[...], preferred_element_type=f32)
               + jnp.dot(ii * t1_ref[...], wd1_ref[...], preferred_element_type=f32)
               + bd_ref[...])
    h = jnp.maximum(0.5 * hs_part + 0.5 * hd_part, 0.0)
    if scale_out:
        outs[0][...] = ii * h[:, :DH]
        outs[1][...] = ii * h[:, DH:]
        outs[2][...] = io * h[:, :DH]
        outs[3][...] = io * h[:, DH:]
    else:
        outs[0][...] = h


def _make_layer(scale_out):
    if scale_out:
        out_specs = [pl.BlockSpec((_BR, DH), lambda i: (i, 0))] * 4
        out_shape = [jax.ShapeDtypeStruct((N, DH), jnp.float32)] * 4
    else:
        out_specs = [pl.BlockSpec((_BR, D), lambda i: (i, 0))]
        out_shape = [jax.ShapeDtypeStruct((N, D), jnp.float32)]
    return pl.pallas_call(
        functools.partial(_layer_body, scale_out),
        grid=(N // _BR,),
        in_specs=[pl.BlockSpec((_BR, DH), lambda i: (i, 0))] * 4
                 + [pl.BlockSpec((_BR, 1), lambda i: (i, 0)),
                    pl.BlockSpec((_BR, 1), lambda i: (i, 0))]
                 + [pl.BlockSpec((DH, D), lambda i: (0, 0))] * 4
                 + [pl.BlockSpec((1, D), lambda i: (0, 0)),
                    pl.BlockSpec((1, D), lambda i: (0, 0))],
        out_specs=out_specs,
        out_shape=out_shape,
    )


_tc_layer_mid = _make_layer(True)
_tc_layer_last = _make_layer(False)

_PBR = 200


def _pool_body(h_ref, b_ref, wl1_ref, bl1_ref, wl2_ref, bl2_ref,
               out_ref, pooled_ref):
    pid = pl.program_id(0)
    ninf = jnp.float32(-jnp.inf)

    @pl.when(pid == 0)
    def _():
        pooled_ref[...] = jnp.full((G, D), ninf, jnp.float32)

    hb = h_ref[...]
    bb = b_ref[...]
    gids = lax.broadcasted_iota(jnp.int32, (G, 1, 1), 0)
    mask = bb[None, :, :] == gids
    v = jnp.where(mask, hb[None, :, :], ninf)
    pooled_ref[...] = jnp.maximum(pooled_ref[...], jnp.max(v, axis=1))

    @pl.when(pid == N // _PBR - 1)
    def _():
        p = pooled_ref[...]
        z = jnp.dot(p, wl1_ref[...], preferred_element_type=jnp.float32) \
            + bl1_ref[...]
        z = jnp.maximum(z, 0.0)
        out_ref[...] = jnp.dot(z, wl2_ref[...],
                               preferred_element_type=jnp.float32) + bl2_ref[...]


_tc_pool = pl.pallas_call(
    _pool_body,
    grid=(N // _PBR,),
    in_specs=[pl.BlockSpec((_PBR, D), lambda i: (i, 0)),
              pl.BlockSpec((_PBR, 1), lambda i: (i, 0)),
              pl.BlockSpec((D, D), lambda i: (0, 0)),
              pl.BlockSpec((1, D), lambda i: (0, 0)),
              pl.BlockSpec((D, D), lambda i: (0, 0)),
              pl.BlockSpec((1, D), lambda i: (0, 0))],
    out_specs=pl.BlockSpec((G, D), lambda i: (0, 0)),
    out_shape=jax.ShapeDtypeStruct((G, D), jnp.float32),
    scratch_shapes=[pltpu.VMEM((G, D), jnp.float32)],
)


def _pad_idx(a, fill):
    pad = jnp.full((EP - E,), fill, jnp.int32)
    return jnp.concatenate([a, pad]).reshape(NT, CH, K)


def kernel(x, edge_index, batch, W1s, b1s, W1d, b1d, W2s, b2s, W2d, b2d,
           W3s, b3s, W3d, b3d, Wl1, bl1, Wl2, bl2):
    row = edge_index[0]
    col = edge_index[1]
    gidx_s = _pad_idx(col, 0)
    sidx_s = _pad_idx(row, DUMMY)
    gidx_t = _pad_idx(row, 0)
    sidx_t = _pad_idx(col, DUMMY)
    zsrc = jnp.zeros((640, DH), jnp.float32)
    ones_rows = jnp.ones((K, _DW), jnp.float32)
    zsrc_d = jnp.zeros((640, _DW), jnp.float32)

    deg_o_p, deg_i_p = _get_sc_degrees()(sidx_s, sidx_t, ones_rows, zsrc_d)
    deg_o = deg_o_p[:N, :1]
    deg_i = deg_i_p[:N, :1]

    io, ii, hs0, hs1, ht0, ht1 = _tc_prep(deg_o, deg_i, x)

    weights = [(W1s, b1s, W1d, b1d), (W2s, b2s, W2d, b2d), (W3s, b3s, W3d, b3d)]
    for l, (ws, bs, wd, bd) in enumerate(weights):
        s0, s1, t0, t1 = _get_sc_agg()(
            hs0, hs1, ht0, ht1, gidx_s, sidx_s, gidx_t, sidx_t, zsrc)
        args = (s0, s1, t0, t1, io, ii, ws[:DH], ws[DH:], wd[:DH], wd[DH:],
                bs.reshape(1, D), bd.reshape(1, D))
        if l < 2:
            hs0, hs1, ht0, ht1 = _tc_layer_mid(*args)
        else:
            h = _tc_layer_last(*args)[0]

    wl1p = jnp.zeros((D, D), jnp.float32).at[:, :5].set(Wl1)
    bl1p = jnp.zeros((1, D), jnp.float32).at[:, :5].set(bl1[None, :])
    wl2p = jnp.zeros((D, D), jnp.float32).at[:5, :1].set(Wl2)
    bl2p = jnp.zeros((1, D), jnp.float32).at[:, :1].set(bl2[None, :])

    out = _tc_pool(h, batch.reshape(N, 1).astype(jnp.int32),
                   wl1p, bl1p, wl2p, bl2p)
    return out[:, :1]
